# baseline (speedup 1.0000x reference)
import functools
import jax
import jax.numpy as jnp
from jax import lax
from jax.experimental import pallas as pl
from jax.experimental.pallas import tpu as pltpu
from jax.experimental.pallas import tpu_sc as plsc

H = 8
D = 128
G1 = 1025
NP = 1040
B = 16
BB = 2
HALF = 512

NW = 32
ROWS = B * NP
RPW = ROWS // NW
CHUNK = 104
NCHUNK = RPW // CHUNK


def _sc_gather_body(hflat, gidx, out_hbm, idx_v, rows_v, sem):
    wid = lax.axis_index("s") * 2 + lax.axis_index("c")
    base = wid * RPW
    pltpu.sync_copy(gidx.at[pl.ds(base, RPW)], idx_v)
    copies = [
        pltpu.async_copy(
            hflat.at[idx_v.at[pl.ds(c * CHUNK, CHUNK)]],
            rows_v.at[pl.ds(c * CHUNK, CHUNK)],
            sem,
        )
        for c in range(NCHUNK)
    ]
    for cp in copies:
        cp.wait()
    pltpu.sync_copy(rows_v, out_hbm.at[pl.ds(base, RPW)])


_sc_gather = functools.partial(
    pl.kernel,
    out_type=jax.ShapeDtypeStruct((ROWS, D), jnp.float32),
    mesh=plsc.VectorSubcoreMesh(core_axis_name="c", subcore_axis_name="s"),
    scratch_types=[
        pltpu.VMEM((RPW,), jnp.int32),
        pltpu.VMEM((RPW, D), jnp.float32),
        pltpu.SemaphoreType.DMA,
    ],
)(_sc_gather_body)


def _tc_body(h_ref, hpre_ref, hpost_ref, sel_ref, WQ_ref, WK_ref,
             W1_ref, b1_ref, W2_ref, b2_ref, W3_ref, b3_ref,
             out_ref, A_ref, G_ref):
    b = pl.program_id(0)

    @pl.when(b == 0)
    def _():
        for hh in range(H):
            A_ref[:, hh * D:(hh + 1) * D] = jnp.dot(
                WQ_ref[hh], WK_ref[hh].T, preferred_element_type=jnp.float32,
                precision=lax.Precision.HIGHEST)
        krow = lax.broadcasted_iota(jnp.int32, (H * D, H), 0) // D
        hcol = lax.broadcasted_iota(jnp.int32, (H * D, H), 1)
        S = jnp.where(krow == hcol, 1.0, 0.0).astype(jnp.float32)
        G_ref[:, :32] = jnp.dot(S, W1_ref[0:8, :],
                                preferred_element_type=jnp.float32,
                                precision=lax.Precision.HIGHEST)
        G_ref[:, 32:] = jnp.dot(S, W1_ref[8:16, :],
                                preferred_element_type=jnp.float32,
                                precision=lax.Precision.HIGHEST)

    h = h_ref[...].reshape(BB * NP, D)
    hpre = hpre_ref[...].reshape(BB * NP, D)
    hpost = hpost_ref[...].reshape(BB * NP, D)
    A = A_ref[:, :]
    U = jnp.dot(h, A, preferred_element_type=jnp.float32)
    V = jnp.dot(hpre, A, preferred_element_type=jnp.float32)
    w = h - hpost
    P_parts = []
    for hh in range(H):
        sl = slice(hh * D, (hh + 1) * D)
        P_parts.append(V[:, sl] * w + U[:, sl] * hpost)
    P = jnp.concatenate(P_parts, axis=1)
    T = jnp.dot(P, G_ref[:, :], preferred_element_type=jnp.float32)
    t_parts = [T[i * NP + 1:i * NP + HALF + 1, :32]
               + T[i * NP + HALF + 1:i * NP + 2 * HALF + 1, 32:]
               for i in range(BB)]
    x = jnp.maximum(
        jnp.concatenate(t_parts, axis=0)
        + jnp.dot(sel_ref[...].reshape(BB * HALF, 4), W1_ref[16:20, :],
                  preferred_element_type=jnp.float32)
        + b1_ref[:, :], 0.0)
    x = jnp.maximum(
        jnp.dot(x, W2_ref[:, :], preferred_element_type=jnp.float32)
        + b2_ref[:, :], 0.0)
    y = (jnp.dot(x, W3_ref[:, :], preferred_element_type=jnp.float32)
         + b3_ref[:, :])
    out_ref[...] = y.reshape(BB, HALF, 1)


def _tc_compute(h_hat, hpre, hpost, sel, W_Q, W_K, W1, b1, W2, b2, W3, b3):
    return pl.pallas_call(
        _tc_body,
        grid=(B // BB,),
        in_specs=[
            pl.BlockSpec((BB, NP, D), lambda b: (b, 0, 0)),
            pl.BlockSpec((BB, NP, D), lambda b: (b, 0, 0)),
            pl.BlockSpec((BB, NP, D), lambda b: (b, 0, 0)),
            pl.BlockSpec((BB, HALF, 4), lambda b: (b, 0, 0)),
            pl.BlockSpec((H, D, D), lambda b: (0, 0, 0)),
            pl.BlockSpec((H, D, D), lambda b: (0, 0, 0)),
            pl.BlockSpec((2 * H + 4, 32), lambda b: (0, 0)),
            pl.BlockSpec((1, 32), lambda b: (0, 0)),
            pl.BlockSpec((32, 32), lambda b: (0, 0)),
            pl.BlockSpec((1, 32), lambda b: (0, 0)),
            pl.BlockSpec((32, 1), lambda b: (0, 0)),
            pl.BlockSpec((1, 1), lambda b: (0, 0)),
        ],
        out_specs=pl.BlockSpec((BB, HALF, 1), lambda b: (b, 0, 0)),
        out_shape=jax.ShapeDtypeStruct((B, HALF, 1), jnp.float32),
        scratch_shapes=[pltpu.VMEM((D, H * D), jnp.float32),
                        pltpu.VMEM((H * D, 64), jnp.float32)],
        compiler_params=pltpu.CompilerParams(
            dimension_semantics=("arbitrary",)),
    )(h_hat, hpre, hpost, sel, W_Q, W_K, W1, b1, W2, b2, W3, b3)


def kernel(h_hat, solution, selection_recent, W_Q, W_K, W1, b1, W2, b2, W3, b3):
    sol = solution.astype(jnp.int32)
    pre = jnp.argsort(sol, axis=1).astype(jnp.int32)

    hpad = jnp.pad(h_hat, ((0, 0), (0, NP - G1), (0, 0)))
    hflat = hpad.reshape(ROWS, D)
    rowoff = (jnp.arange(B, dtype=jnp.int32) * NP)[:, None]
    gpre = (jnp.pad(pre, ((0, 0), (0, NP - G1))) + rowoff).reshape(-1)
    gpost = (jnp.pad(sol, ((0, 0), (0, NP - G1))) + rowoff).reshape(-1)

    hpost = _sc_gather(hflat, gpost).reshape(B, NP, D)
    hpre = _sc_gather(hflat, gpre).reshape(B, NP, D)

    sel = jnp.transpose(selection_recent, (0, 2, 1))
    out = _tc_compute(hpad, hpre, hpost, sel, W_Q, W_K,
                      W1, b1.reshape(1, -1), W2, b2.reshape(1, -1),
                      W3, b3.reshape(1, 1))
    return out.reshape(B, HALF)


# speedup vs baseline: 1.0401x; 1.0401x over previous
import functools
import jax
import jax.numpy as jnp
from jax import lax
from jax.experimental import pallas as pl
from jax.experimental.pallas import tpu as pltpu
from jax.experimental.pallas import tpu_sc as plsc

H = 8
D = 128
G1 = 1025
NP = 1040
B = 16
BB = 2
HALF = 512

NW = 32
ROWS = B * NP
RPW = ROWS // NW
CHUNK = 104
NCHUNK = RPW // CHUNK


def _sc_gather_body(hflat, gpre, gpost, opre, opost, idx_v, rows_v, sem):
    wid = lax.axis_index("s") * 2 + lax.axis_index("c")
    base = wid * RPW
    for idx_hbm, out_hbm in ((gpost, opost), (gpre, opre)):
        pltpu.sync_copy(idx_hbm.at[pl.ds(base, RPW)], idx_v)
        copies = [
            pltpu.async_copy(
                hflat.at[idx_v.at[pl.ds(c * CHUNK, CHUNK)]],
                rows_v.at[pl.ds(c * CHUNK, CHUNK)],
                sem,
            )
            for c in range(NCHUNK)
        ]
        for cp in copies:
            cp.wait()
        pltpu.sync_copy(rows_v, out_hbm.at[pl.ds(base, RPW)])


_sc_gather = functools.partial(
    pl.kernel,
    out_type=[
        jax.ShapeDtypeStruct((ROWS, D), jnp.float32),
        jax.ShapeDtypeStruct((ROWS, D), jnp.float32),
    ],
    mesh=plsc.VectorSubcoreMesh(core_axis_name="c", subcore_axis_name="s"),
    scratch_types=[
        pltpu.VMEM((RPW,), jnp.int32),
        pltpu.VMEM((RPW, D), jnp.float32),
        pltpu.SemaphoreType.DMA,
    ],
)(_sc_gather_body)


def _tc_body(h_ref, hpre_ref, hpost_ref, sel_ref, WQ_ref, WK_ref,
             W1_ref, b1_ref, W2_ref, b2_ref, W3_ref, b3_ref,
             out_ref, A_ref, G_ref):
    b = pl.program_id(0)

    @pl.when(b == 0)
    def _():
        for hh in range(H):
            A_ref[:, hh * D:(hh + 1) * D] = jnp.dot(
                WQ_ref[hh], WK_ref[hh].T, preferred_element_type=jnp.float32,
                precision=lax.Precision.HIGHEST)
        krow = lax.broadcasted_iota(jnp.int32, (H * D, H), 0) // D
        hcol = lax.broadcasted_iota(jnp.int32, (H * D, H), 1)
        S = jnp.where(krow == hcol, 1.0, 0.0).astype(jnp.float32)
        G_ref[:, :32] = jnp.dot(S, W1_ref[0:8, :],
                                preferred_element_type=jnp.float32,
                                precision=lax.Precision.HIGHEST)
        G_ref[:, 32:] = jnp.dot(S, W1_ref[8:16, :],
                                preferred_element_type=jnp.float32,
                                precision=lax.Precision.HIGHEST)

    h = h_ref[...].reshape(BB * NP, D)
    hpre = hpre_ref[...].reshape(BB * NP, D)
    hpost = hpost_ref[...].reshape(BB * NP, D)
    A = A_ref[:, :]
    U = jnp.dot(h, A, preferred_element_type=jnp.float32)
    V = jnp.dot(hpre, A, preferred_element_type=jnp.float32)
    w = h - hpost
    P_parts = []
    for hh in range(H):
        sl = slice(hh * D, (hh + 1) * D)
        P_parts.append(V[:, sl] * w + U[:, sl] * hpost)
    P = jnp.concatenate(P_parts, axis=1)
    T = jnp.dot(P, G_ref[:, :], preferred_element_type=jnp.float32)
    t_parts = [T[i * NP + 1:i * NP + HALF + 1, :32]
               + T[i * NP + HALF + 1:i * NP + 2 * HALF + 1, 32:]
               for i in range(BB)]
    x = jnp.maximum(
        jnp.concatenate(t_parts, axis=0)
        + jnp.dot(sel_ref[...].reshape(BB * HALF, 4), W1_ref[16:20, :],
                  preferred_element_type=jnp.float32)
        + b1_ref[:, :], 0.0)
    x = jnp.maximum(
        jnp.dot(x, W2_ref[:, :], preferred_element_type=jnp.float32)
        + b2_ref[:, :], 0.0)
    y = (jnp.dot(x, W3_ref[:, :], preferred_element_type=jnp.float32)
         + b3_ref[:, :])
    out_ref[...] = y.reshape(1, 1, BB * HALF)


def _tc_compute(h_hat, hpre, hpost, sel, W_Q, W_K, W1, b1, W2, b2, W3, b3):
    return pl.pallas_call(
        _tc_body,
        grid=(B // BB,),
        in_specs=[
            pl.BlockSpec((BB, NP, D), lambda b: (b, 0, 0)),
            pl.BlockSpec((BB, NP, D), lambda b: (b, 0, 0)),
            pl.BlockSpec((BB, NP, D), lambda b: (b, 0, 0)),
            pl.BlockSpec((BB, HALF, 4), lambda b: (b, 0, 0)),
            pl.BlockSpec((H, D, D), lambda b: (0, 0, 0)),
            pl.BlockSpec((H, D, D), lambda b: (0, 0, 0)),
            pl.BlockSpec((2 * H + 4, 32), lambda b: (0, 0)),
            pl.BlockSpec((1, 32), lambda b: (0, 0)),
            pl.BlockSpec((32, 32), lambda b: (0, 0)),
            pl.BlockSpec((1, 32), lambda b: (0, 0)),
            pl.BlockSpec((32, 1), lambda b: (0, 0)),
            pl.BlockSpec((1, 1), lambda b: (0, 0)),
        ],
        out_specs=pl.BlockSpec((1, 1, BB * HALF), lambda b: (b, 0, 0)),
        out_shape=jax.ShapeDtypeStruct((B // BB, 1, BB * HALF), jnp.float32),
        scratch_shapes=[pltpu.VMEM((D, H * D), jnp.float32),
                        pltpu.VMEM((H * D, 64), jnp.float32)],
        compiler_params=pltpu.CompilerParams(
            dimension_semantics=("arbitrary",)),
    )(h_hat, hpre, hpost, sel, W_Q, W_K, W1, b1, W2, b2, W3, b3)


def kernel(h_hat, solution, selection_recent, W_Q, W_K, W1, b1, W2, b2, W3, b3):
    sol = solution.astype(jnp.int32)
    pre = jnp.argsort(sol, axis=1).astype(jnp.int32)

    hpad = jnp.pad(h_hat, ((0, 0), (0, NP - G1), (0, 0)))
    hflat = hpad.reshape(ROWS, D)
    rowoff = (jnp.arange(B, dtype=jnp.int32) * NP)[:, None]
    gpre = (jnp.pad(pre, ((0, 0), (0, NP - G1))) + rowoff).reshape(-1)
    gpost = (jnp.pad(sol, ((0, 0), (0, NP - G1))) + rowoff).reshape(-1)

    hpre_flat, hpost_flat = _sc_gather(hflat, gpre, gpost)
    hpre = hpre_flat.reshape(B, NP, D)
    hpost = hpost_flat.reshape(B, NP, D)

    sel = jnp.transpose(selection_recent, (0, 2, 1))
    out = _tc_compute(hpad, hpre, hpost, sel, W_Q, W_K,
                      W1, b1.reshape(1, -1), W2, b2.reshape(1, -1),
                      W3, b3.reshape(1, 1))
    return out.reshape(B, HALF)
